# Initial kernel scaffold; baseline (speedup 1.0000x reference)
import jax
import jax.numpy as jnp
from jax.experimental import pallas as pl


def kernel(x, separated_src_idx, unique):
    raise NotImplementedError("write your pallas kernel here")


# speedup vs baseline: 3.0402x; 3.0402x over previous
import functools

import jax
import jax.numpy as jnp
from jax import lax
from jax.experimental import pallas as pl
from jax.experimental.pallas import tpu as pltpu
from jax.experimental.pallas import tpu_sc as plsc

V = 10242
F3 = 3 * 20480
N_UNIQUE = 30720
B = 8
NCH = 64
K = B * NCH

NC = 2
NS = 16
L = 16
NW = NC * NS
ROWS_PER_W = N_UNIQUE // NW
CHUNK = 64
NCHUNK = ROWS_PER_W // CHUNK


def _tc_parts_body(x_ref, p1_ref, tbl_ref):
    xb = x_ref[0]
    p1_ref[0] = (xb[0:64] + xb[64:128]) * 0.5
    tbl_ref[0] = (xb[128:192] + xb[192:256]) * 0.25


def _tc_parts(x):
    return pl.pallas_call(
        _tc_parts_body,
        grid=(B,),
        in_specs=[pl.BlockSpec((1, 4 * NCH, V), lambda b: (b, 0, 0))],
        out_specs=[
            pl.BlockSpec((1, NCH, V), lambda b: (b, 0, 0)),
            pl.BlockSpec((1, NCH, V), lambda b: (b, 0, 0)),
        ],
        out_shape=[
            jax.ShapeDtypeStruct((B, NCH, V), jnp.float32),
            jax.ShapeDtypeStruct((B, NCH, V), jnp.float32),
        ],
    )(x)


def _sc_gather(tbl_t, sep_flat, unique):
    mesh = plsc.VectorSubcoreMesh(core_axis_name="c", subcore_axis_name="s")

    @functools.partial(
        pl.kernel,
        out_type=jax.ShapeDtypeStruct((N_UNIQUE, K), jnp.float32),
        mesh=mesh,
        scratch_types=[
            pltpu.VMEM((CHUNK,), jnp.int32),
            pltpu.VMEM((CHUNK,), jnp.int32),
            pltpu.VMEM((CHUNK,), jnp.int32),
            pltpu.VMEM((CHUNK,), jnp.int32),
            pltpu.VMEM((CHUNK,), jnp.int32),
            pltpu.VMEM((CHUNK, K), jnp.float32),
            pltpu.VMEM((CHUNK, K), jnp.float32),
            pltpu.SemaphoreType.DMA,
            pltpu.SemaphoreType.DMA,
        ],
    )
    def k(tbl_hbm, sep_hbm, uq_hbm, out_hbm,
          uq_v, ia_v, ib_v, a_v, b_v, rows_a, rows_b, sem1, sem2):
        wid = lax.axis_index("s") * NC + lax.axis_index("c")
        base0 = wid * ROWS_PER_W

        def chunk_body(ci, carry):
            base = base0 + ci * CHUNK
            pltpu.sync_copy(uq_hbm.at[pl.ds(base, CHUNK)], uq_v)
            for j in range(CHUNK // L):
                u2 = uq_v[pl.ds(j * L, L)] * 2
                ia_v[pl.ds(j * L, L)] = u2
                ib_v[pl.ds(j * L, L)] = u2 + 1
            ga = pltpu.async_copy(sep_hbm.at[ia_v], a_v, sem1)
            gb = pltpu.async_copy(sep_hbm.at[ib_v], b_v, sem2)
            ga.wait()
            gb.wait()
            ca = pltpu.async_copy(tbl_hbm.at[a_v], rows_a, sem1)
            cb = pltpu.async_copy(tbl_hbm.at[b_v], rows_b, sem2)
            ca.wait()
            cb.wait()

            def add_row(r, c):
                for j in range(K // L):
                    plsc.addupdate(rows_a.at[r, pl.ds(j * L, L)],
                                   rows_b[r, pl.ds(j * L, L)])
                return c

            lax.fori_loop(0, CHUNK, add_row, 0, unroll=2)
            pltpu.sync_copy(rows_a, out_hbm.at[pl.ds(base, CHUNK)])
            return carry

        lax.fori_loop(0, NCHUNK, chunk_body, 0)

    return k(tbl_t, sep_flat, unique)


def kernel(x, separated_src_idx, unique):
    part1, tbl = _tc_parts(x)
    tbl_t = tbl.reshape(K, V).T
    sep_flat = separated_src_idx.reshape(F3 * 2)
    outg = _sc_gather(tbl_t, sep_flat, unique)
    face = outg.T.reshape(B, NCH, N_UNIQUE)
    return jnp.concatenate([part1, face], axis=2)


# speedup vs baseline: 3.3671x; 1.1075x over previous
import functools

import jax
import jax.numpy as jnp
from jax import lax
from jax.experimental import pallas as pl
from jax.experimental.pallas import tpu as pltpu
from jax.experimental.pallas import tpu_sc as plsc

V = 10242
F3 = 3 * 20480
N_UNIQUE = 30720
B = 8
NCH = 64
K = B * NCH
OUTW = V + N_UNIQUE

NC = 2
NS = 16
L = 16
NW = NC * NS
JPT = N_UNIQUE // NW
CHUNK = 64
NCHUNK = JPT // CHUNK


def _tbl_body(x_ref, tbl_ref):
    xb = x_ref[...]
    t = jnp.concatenate([
        (xb[0, 0:64] + xb[0, 64:128]),
        (xb[1, 0:64] + xb[1, 64:128]),
    ], axis=0) * 0.25
    tbl_ref[...] = t.T


def _tc_tables(x):
    return pl.pallas_call(
        _tbl_body,
        grid=(B // 2,),
        in_specs=[pl.BlockSpec((2, 128, V), lambda b: (b, 1, 0))],
        out_specs=pl.BlockSpec((V, 128), lambda b: (0, b)),
        out_shape=jax.ShapeDtypeStruct((V, K), jnp.float32),
    )(x)


def _sc_gather(tbl, sep_flat, unique):
    mesh = plsc.VectorSubcoreMesh(core_axis_name="c", subcore_axis_name="s")

    @functools.partial(
        pl.kernel,
        out_type=jax.ShapeDtypeStruct((N_UNIQUE, K), jnp.float32),
        mesh=mesh,
        scratch_types=[
            pltpu.VMEM((JPT,), jnp.int32),
            pltpu.VMEM((JPT,), jnp.int32),
            pltpu.VMEM((CHUNK,), jnp.int32),
            pltpu.VMEM((CHUNK,), jnp.int32),
            pltpu.VMEM((CHUNK,), jnp.int32),
            pltpu.VMEM((CHUNK, K), jnp.float32),
            pltpu.VMEM((CHUNK, K), jnp.float32),
            pltpu.SemaphoreType.DMA,
            pltpu.SemaphoreType.DMA,
        ],
    )
    def k(tbl_hbm, sep_hbm, uq_hbm, out_hbm,
          ia_all, ib_all, uq_v, u2a, u2b, rows_a, rows_b, sem1, sem2):
        wid = lax.axis_index("s") * NC + lax.axis_index("c")
        jbase = wid * JPT

        for t in range(NCHUNK):
            pltpu.sync_copy(uq_hbm.at[pl.ds(jbase + t * CHUNK, CHUNK)], uq_v)
            for j in range(CHUNK // L):
                u2 = uq_v[pl.ds(j * L, L)] * 2
                u2a[pl.ds(j * L, L)] = u2
                u2b[pl.ds(j * L, L)] = u2 + 1
            ga = pltpu.async_copy(sep_hbm.at[u2a],
                                  ia_all.at[pl.ds(t * CHUNK, CHUNK)], sem1)
            gb = pltpu.async_copy(sep_hbm.at[u2b],
                                  ib_all.at[pl.ds(t * CHUNK, CHUNK)], sem2)
            ga.wait()
            gb.wait()

        for t in range(NCHUNK):
            j0 = jbase + t * CHUNK
            ia = ia_all.at[pl.ds(t * CHUNK, CHUNK)]
            ib = ib_all.at[pl.ds(t * CHUNK, CHUNK)]
            ca = pltpu.async_copy(tbl_hbm.at[ia], rows_a, sem1)
            cb = pltpu.async_copy(tbl_hbm.at[ib], rows_b, sem2)
            ca.wait()
            cb.wait()

            def add_row(r, carry):
                for u in range(K // L):
                    plsc.addupdate(rows_a.at[r, pl.ds(u * L, L)],
                                   rows_b[r, pl.ds(u * L, L)])
                return carry

            lax.fori_loop(0, CHUNK, add_row, 0, unroll=2)
            pltpu.sync_copy(rows_a, out_hbm.at[pl.ds(j0, CHUNK), :])

    return k(tbl, sep_flat, unique)


JH = N_UNIQUE // 2


def _asm_body(x_ref, outg_ref, out_ref):
    h = pl.program_id(1)
    parity = pl.program_id(0) % 2
    sub = JH // 4

    @pl.when(h == 0)
    def _():
        xb = x_ref[0]
        out_ref[0, :, 0:V] = (xb[0:64] + xb[64:128]) * 0.5

    for par in range(2):
        r0 = par * 64

        @pl.when((h == 0) & (parity == par))
        def _():
            for c in range(4):
                tt = outg_ref[c * sub:(c + 1) * sub, :].T
                out_ref[0, :, V + c * sub:V + (c + 1) * sub] = (
                    tt[r0:r0 + 64, :])

        @pl.when((h == 1) & (parity == par))
        def _():
            for c in range(4):
                tt = outg_ref[c * sub:(c + 1) * sub, :].T
                out_ref[0, :, V + JH + c * sub:V + JH + (c + 1) * sub] = (
                    tt[r0:r0 + 64, :])


def _tc_assemble(x, outg):
    return pl.pallas_call(
        _asm_body,
        grid=(B, 2),
        in_specs=[
            pl.BlockSpec((1, 128, V), lambda b, h: (b, 0, 0)),
            pl.BlockSpec((JH, 128), lambda b, h: (h, b // 2)),
        ],
        out_specs=pl.BlockSpec((1, NCH, OUTW), lambda b, h: (b, 0, 0)),
        out_shape=jax.ShapeDtypeStruct((B, NCH, OUTW), jnp.float32),
    )(x, outg)


def kernel(x, separated_src_idx, unique):
    tbl = _tc_tables(x)
    sep_flat = separated_src_idx.reshape(F3 * 2)
    outg = _sc_gather(tbl, sep_flat, unique)
    return _tc_assemble(x, outg)
